# Initial kernel scaffold; baseline (speedup 1.0000x reference)
import jax
import jax.numpy as jnp
from jax.experimental import pallas as pl


def kernel(vectors, node_feats, radial_embedding, senders, receivers, W0, W1, W2, W3):
    raise NotImplementedError("write your pallas kernel here")


# speedup vs baseline: 1.1090x; 1.1090x over previous
import functools
import math

import jax
import jax.numpy as jnp
from jax import lax
from jax.experimental import pallas as pl
from jax.experimental.pallas import tpu as pltpu
from jax.experimental.pallas import tpu_sc as plsc

N_NODES = 10000
N_EDGES = 320000
D_FEAT = 128
AVG_NUM_NEIGHBORS = 32.0


_B = 1000


def _coef_body(vec_ref, rad_ref, w0, w1, w2, w3, coef_ref, shb_ref):
    prec = jax.lax.Precision.HIGHEST
    x = rad_ref[...]
    h = jax.nn.silu(jnp.dot(x, w0[...], precision=prec,
                            preferred_element_type=jnp.float32) * (8.0 ** -0.5))
    h = jax.nn.silu(jnp.dot(h, w1[...], precision=prec,
                            preferred_element_type=jnp.float32) * 0.125)
    h = jax.nn.silu(jnp.dot(h, w2[...], precision=prec,
                            preferred_element_type=jnp.float32) * 0.125)
    mix = jnp.dot(h, w3[...], precision=prec,
                  preferred_element_type=jnp.float32)
    mix = mix * (0.125 / math.sqrt(AVG_NUM_NEIGHBORS))
    coef_ref[0] = mix[:, :D_FEAT]
    coef_ref[1] = mix[:, D_FEAT:]

    v = vec_ref[...]
    inv = jax.lax.rsqrt(jnp.sum(v * v, axis=1, keepdims=True) + 1e-12)
    sh = v * inv * math.sqrt(3.0)
    shb_ref[0] = jnp.ones((_B, 16), jnp.float32)
    for k in range(3):
        shb_ref[k + 1] = jnp.broadcast_to(sh[:, k:k + 1], (_B, 16))


def _coef_fn(vectors, radial, W0, W1, W2, W3):
    grid = (N_EDGES // _B,)
    return pl.pallas_call(
        _coef_body,
        grid=grid,
        in_specs=[
            pl.BlockSpec((_B, 3), lambda i: (i, 0)),
            pl.BlockSpec((_B, 8), lambda i: (i, 0)),
            pl.BlockSpec((8, 64), lambda i: (0, 0)),
            pl.BlockSpec((64, 64), lambda i: (0, 0)),
            pl.BlockSpec((64, 64), lambda i: (0, 0)),
            pl.BlockSpec((64, 256), lambda i: (0, 0)),
        ],
        out_specs=[
            pl.BlockSpec((2, _B, 128), lambda i: (0, i, 0)),
            pl.BlockSpec((4, _B, 16), lambda i: (0, i, 0)),
        ],
        out_shape=[
            jax.ShapeDtypeStruct((2, N_EDGES, 128), jnp.float32),
            jax.ShapeDtypeStruct((4, N_EDGES, 16), jnp.float32),
        ],
    )(vectors, radial, W0, W1, W2, W3)


_NS = 16
_EPT = N_EDGES // _NS
_C = 80
_NCHUNK = _EPT // _C
_NP = 10240
_RPT = _NP // _NS


def _sc_body(nf_hbm, snd_hbm, rcv_hbm, coef_hbm, shb_hbm, out_hbm,
             acc, snd_v, rcv_v, f_v, c_v, sh_v, m_v, sem):
    cid = lax.axis_index("c")
    sid = lax.axis_index("s")
    tile_start = sid * _EPT
    row0 = sid * _RPT

    for r in range(2):
        b = cid * 2 + r
        jc = jnp.minimum(b, 1)

        def _zb(i, carry):
            for j in range(8):
                m_v[i, pl.ds(j * 16, 16)] = jnp.zeros((16,), jnp.float32)
            return carry
        lax.fori_loop(0, _C, _zb, 0)
        for z in range(_RPT // _C):
            pltpu.sync_copy(m_v, acc.at[pl.ds(row0 + z * _C, _C), :])
        plsc.subcore_barrier()

        def _chunk(ci, carry):
            base = tile_start + ci * _C
            pltpu.sync_copy(snd_hbm.at[pl.ds(base, _C)], snd_v)
            pltpu.sync_copy(rcv_hbm.at[pl.ds(base, _C)], rcv_v)
            pltpu.sync_copy(coef_hbm.at[jc, pl.ds(base, _C), :], c_v)
            pltpu.sync_copy(shb_hbm.at[b, pl.ds(base, _C), :], sh_v)
            pltpu.async_copy(nf_hbm.at[snd_v], f_v, sem).wait()

            def _edge(i, c2):
                s = sh_v[i, :]
                for j in range(8):
                    sl = pl.ds(j * 16, 16)
                    m_v[i, sl] = f_v[i, sl] * c_v[i, sl] * s
                return c2
            lax.fori_loop(0, _C, _edge, 0)

            pltpu.sync_copy(m_v, acc.at[rcv_v], add=True)
            return carry
        lax.fori_loop(0, _NCHUNK, _chunk, 0)
        plsc.subcore_barrier()

        for z in range(_RPT // _C):
            rr = row0 + z * _C
            pltpu.sync_copy(acc.at[pl.ds(rr, _C), :],
                            out_hbm.at[b, pl.ds(rr, _C), :])


def _sc_fn(node_feats, senders, receivers, coef2, shb):
    mesh = plsc.VectorSubcoreMesh(core_axis_name="c", subcore_axis_name="s")
    kern = functools.partial(
        pl.kernel,
        out_type=jax.ShapeDtypeStruct((4, _NP, 128), jnp.float32),
        mesh=mesh,
        scratch_types=[
            pltpu.VMEM_SHARED((_NP, 128), jnp.float32),
            pltpu.VMEM((_C,), jnp.int32),
            pltpu.VMEM((_C,), jnp.int32),
            pltpu.VMEM((_C, 128), jnp.float32),
            pltpu.VMEM((_C, 128), jnp.float32),
            pltpu.VMEM((_C, 16), jnp.float32),
            pltpu.VMEM((_C, 128), jnp.float32),
            pltpu.SemaphoreType.DMA,
        ],
    )(_sc_body)
    return kern(node_feats, senders, receivers, coef2, shb)


def kernel(vectors, node_feats, radial_embedding, senders, receivers,
           W0, W1, W2, W3):
    coef2, shb = _coef_fn(vectors, radial_embedding, W0, W1, W2, W3)
    out4 = _sc_fn(node_feats, senders, receivers, coef2, shb)[:, :N_NODES]
    out_s = out4[0]
    out_v = out4[1:].transpose(1, 2, 0).reshape(N_NODES, 3 * D_FEAT)
    return jnp.concatenate([out_s, out_v], axis=1)


# speedup vs baseline: 1.4427x; 1.3009x over previous
import functools
import math

import jax
import jax.numpy as jnp
from jax import lax
from jax.experimental import pallas as pl
from jax.experimental.pallas import tpu as pltpu
from jax.experimental.pallas import tpu_sc as plsc

N_NODES = 10000
N_EDGES = 320000
D_FEAT = 128
AVG_NUM_NEIGHBORS = 32.0


_B = 1000


def _coef_body(vec_ref, rad_ref, w0, w1, w2, w3, coef_ref, shb_ref):
    prec = jax.lax.Precision.HIGHEST
    x = rad_ref[...]
    h = jax.nn.silu(jnp.dot(x, w0[...], precision=prec,
                            preferred_element_type=jnp.float32) * (8.0 ** -0.5))
    h = jax.nn.silu(jnp.dot(h, w1[...], precision=prec,
                            preferred_element_type=jnp.float32) * 0.125)
    h = jax.nn.silu(jnp.dot(h, w2[...], precision=prec,
                            preferred_element_type=jnp.float32) * 0.125)
    mix = jnp.dot(h, w3[...], precision=prec,
                  preferred_element_type=jnp.float32)
    mix = mix * (0.125 / math.sqrt(AVG_NUM_NEIGHBORS))
    coef_ref[0] = mix[:, :D_FEAT]
    coef_ref[1] = mix[:, D_FEAT:]

    v = vec_ref[...]
    inv = jax.lax.rsqrt(jnp.sum(v * v, axis=1, keepdims=True) + 1e-12)
    sh = v * inv * math.sqrt(3.0)
    shb_ref[0] = jnp.ones((_B, 16), jnp.float32)
    for k in range(3):
        shb_ref[k + 1] = jnp.broadcast_to(sh[:, k:k + 1], (_B, 16))


def _coef_fn(vectors, radial, W0, W1, W2, W3):
    grid = (N_EDGES // _B,)
    return pl.pallas_call(
        _coef_body,
        grid=grid,
        in_specs=[
            pl.BlockSpec((_B, 3), lambda i: (i, 0)),
            pl.BlockSpec((_B, 8), lambda i: (i, 0)),
            pl.BlockSpec((8, 64), lambda i: (0, 0)),
            pl.BlockSpec((64, 64), lambda i: (0, 0)),
            pl.BlockSpec((64, 64), lambda i: (0, 0)),
            pl.BlockSpec((64, 256), lambda i: (0, 0)),
        ],
        out_specs=[
            pl.BlockSpec((2, _B, 128), lambda i: (0, i, 0)),
            pl.BlockSpec((4, _B, 16), lambda i: (0, i, 0)),
        ],
        out_shape=[
            jax.ShapeDtypeStruct((2, N_EDGES, 128), jnp.float32),
            jax.ShapeDtypeStruct((4, N_EDGES, 16), jnp.float32),
        ],
    )(vectors, radial, W0, W1, W2, W3)


_NS = 16
_EPT = N_EDGES // _NS
_C = 40
_NCHUNK = _EPT // _C
_NP = 10240
_RPT = _NP // _NS


def _sc_body(nf_hbm, snd_hbm, rcv_hbm, coef_hbm, shb_hbm, out_hbm,
             acc, snd_v, rcv_v, f_v, c_v, sh_v, m_v,
             sem_in, sem_g, sem_sc):
    cid = lax.axis_index("c")
    sid = lax.axis_index("s")
    tile_start = sid * _EPT
    row0 = sid * _RPT
    n = _NCHUNK

    for r in range(2):
        b = cid * 2 + r
        jc = jnp.minimum(b, 1)

        def _issue_inputs(ci, k):
            i4, i2 = k % 4, k % 2
            base = tile_start + ci * _C
            pltpu.async_copy(snd_hbm.at[pl.ds(base, _C)], snd_v.at[i4], sem_in.at[i4])
            pltpu.async_copy(rcv_hbm.at[pl.ds(base, _C)], rcv_v.at[i4], sem_in.at[i4])
            pltpu.async_copy(coef_hbm.at[jc, pl.ds(base, _C), :], c_v.at[i2], sem_in.at[i4])
            pltpu.async_copy(shb_hbm.at[b, pl.ds(base, _C), :], sh_v.at[i2], sem_in.at[i4])

        def _wait_inputs(k):
            i4, i2 = k % 4, k % 2
            pltpu.make_async_copy(snd_hbm.at[pl.ds(0, _C)], snd_v.at[i4], sem_in.at[i4]).wait()
            pltpu.make_async_copy(rcv_hbm.at[pl.ds(0, _C)], rcv_v.at[i4], sem_in.at[i4]).wait()
            pltpu.make_async_copy(coef_hbm.at[0, pl.ds(0, _C), :], c_v.at[i2], sem_in.at[i4]).wait()
            pltpu.make_async_copy(shb_hbm.at[0, pl.ds(0, _C), :], sh_v.at[i2], sem_in.at[i4]).wait()

        def _issue_gather(k):
            i4, i2 = k % 4, k % 2
            pltpu.async_copy(nf_hbm.at[snd_v.at[i4]], f_v.at[i2], sem_g.at[i2])

        def _wait_gather(k):
            i4, i2 = k % 4, k % 2
            pltpu.make_async_copy(nf_hbm.at[snd_v.at[i4]], f_v.at[i2], sem_g.at[i2]).wait()

        def _issue_scatter(k):
            i4, i2 = k % 4, k % 2
            pltpu.async_copy(m_v.at[i2], acc.at[rcv_v.at[i4]], sem_sc.at[i2], add=True)

        def _wait_scatter(k):
            i4, i2 = k % 4, k % 2
            pltpu.make_async_copy(m_v.at[i2], acc.at[rcv_v.at[i4]], sem_sc.at[i2]).wait()

        def _zb(i, carry):
            for j in range(8):
                m_v[0, i, pl.ds(j * 16, 16)] = jnp.zeros((16,), jnp.float32)
            return carry
        lax.fori_loop(0, _C, _zb, 0)
        for z in range(_RPT // _C):
            pltpu.sync_copy(m_v.at[0], acc.at[pl.ds(row0 + z * _C, _C), :])
        plsc.subcore_barrier()

        _issue_inputs(0, 0)
        _issue_inputs(1, 1)
        _wait_inputs(0)
        _issue_gather(0)

        def _outer(ci4, carry):
            for k in range(4):
                ci = ci4 * 4 + k

                @pl.when(ci + 1 < n)
                def _():
                    _wait_inputs(k + 1)
                    _issue_gather(k + 1)

                _wait_gather(k)

                @pl.when(ci >= 2)
                def _():
                    _wait_scatter(k)

                i2 = k % 2

                def _edge(i, c2):
                    s = sh_v[i2, i, :]
                    for j in range(8):
                        sl = pl.ds(j * 16, 16)
                        m_v[i2, i, sl] = f_v[i2, i, sl] * c_v[i2, i, sl] * s
                    return c2
                lax.fori_loop(0, _C, _edge, 0)

                _issue_scatter(k)

                @pl.when(ci + 2 < n)
                def _():
                    _issue_inputs(ci + 2, k + 2)
            return carry
        lax.fori_loop(0, n // 4, _outer, 0)

        _wait_scatter(0)
        _wait_scatter(1)
        plsc.subcore_barrier()

        for z in range(_RPT // _C):
            rr = row0 + z * _C
            pltpu.sync_copy(acc.at[pl.ds(rr, _C), :],
                            out_hbm.at[b, pl.ds(rr, _C), :])


def _sc_fn(node_feats, senders, receivers, coef2, shb):
    mesh = plsc.VectorSubcoreMesh(core_axis_name="c", subcore_axis_name="s")
    kern = functools.partial(
        pl.kernel,
        out_type=jax.ShapeDtypeStruct((4, _NP, 128), jnp.float32),
        mesh=mesh,
        scratch_types=[
            pltpu.VMEM_SHARED((_NP, 128), jnp.float32),
            pltpu.VMEM((4, _C), jnp.int32),
            pltpu.VMEM((4, _C), jnp.int32),
            pltpu.VMEM((2, _C, 128), jnp.float32),
            pltpu.VMEM((2, _C, 128), jnp.float32),
            pltpu.VMEM((2, _C, 16), jnp.float32),
            pltpu.VMEM((2, _C, 128), jnp.float32),
            pltpu.SemaphoreType.DMA((4,)),
            pltpu.SemaphoreType.DMA((2,)),
            pltpu.SemaphoreType.DMA((2,)),
        ],
    )(_sc_body)
    return kern(node_feats, senders, receivers, coef2, shb)


def kernel(vectors, node_feats, radial_embedding, senders, receivers,
           W0, W1, W2, W3):
    coef2, shb = _coef_fn(vectors, radial_embedding, W0, W1, W2, W3)
    out4 = _sc_fn(node_feats, senders, receivers, coef2, shb)[:, :N_NODES]
    out_s = out4[0]
    out_v = out4[1:].transpose(1, 2, 0).reshape(N_NODES, 3 * D_FEAT)
    return jnp.concatenate([out_s, out_v], axis=1)


# speedup vs baseline: 1.7104x; 1.1856x over previous
import functools
import math

import jax
import jax.numpy as jnp
from jax import lax
from jax.experimental import pallas as pl
from jax.experimental.pallas import tpu as pltpu
from jax.experimental.pallas import tpu_sc as plsc

N_NODES = 10000
N_EDGES = 320000
D_FEAT = 128
AVG_NUM_NEIGHBORS = 32.0


_B = 1000


def _coef_body(vec_ref, rad_ref, w0, w1, w2, w3, w4_ref):
    prec = jax.lax.Precision.HIGHEST
    x = rad_ref[...]
    h = jax.nn.silu(jnp.dot(x, w0[...], precision=prec,
                            preferred_element_type=jnp.float32) * (8.0 ** -0.5))
    h = jax.nn.silu(jnp.dot(h, w1[...], precision=prec,
                            preferred_element_type=jnp.float32) * 0.125)
    h = jax.nn.silu(jnp.dot(h, w2[...], precision=prec,
                            preferred_element_type=jnp.float32) * 0.125)
    mix = jnp.dot(h, w3[...], precision=prec,
                  preferred_element_type=jnp.float32)
    mix = mix * (0.125 / math.sqrt(AVG_NUM_NEIGHBORS))

    v = vec_ref[...]
    inv = jax.lax.rsqrt(jnp.sum(v * v, axis=1, keepdims=True) + 1e-12)
    sh = v * inv * math.sqrt(3.0)

    cv = mix[:, D_FEAT:]
    w4_ref[0] = mix[:, :D_FEAT]
    for k in range(3):
        w4_ref[k + 1] = cv * sh[:, k:k + 1]


def _coef_fn(vectors, radial, W0, W1, W2, W3):
    grid = (N_EDGES // _B,)
    return pl.pallas_call(
        _coef_body,
        grid=grid,
        in_specs=[
            pl.BlockSpec((_B, 3), lambda i: (i, 0)),
            pl.BlockSpec((_B, 8), lambda i: (i, 0)),
            pl.BlockSpec((8, 64), lambda i: (0, 0)),
            pl.BlockSpec((64, 64), lambda i: (0, 0)),
            pl.BlockSpec((64, 64), lambda i: (0, 0)),
            pl.BlockSpec((64, 256), lambda i: (0, 0)),
        ],
        out_specs=pl.BlockSpec((4, _B, 128), lambda i: (0, i, 0)),
        out_shape=jax.ShapeDtypeStruct((4, N_EDGES, 128), jnp.float32),
    )(vectors, radial, W0, W1, W2, W3)


_NS = 16
_EPT = N_EDGES // _NS
_C = 80
_NCHUNK = _EPT // _C
_NP = 10240
_RPT = _NP // _NS


def _sc_body(nf_hbm, snd_hbm, rcv_hbm, w4_hbm, out_hbm,
             acc, snd_v, rcv_v, fm_v, w_v, sem_in, sem_g, sem_sc):
    cid = lax.axis_index("c")
    sid = lax.axis_index("s")
    chunk0 = sid * _NCHUNK
    row0 = sid * _RPT
    n = _NCHUNK

    for r in range(2):
        b = cid * 2 + r

        def _issue_inputs(ci, k):
            i4, i2 = k % 4, k % 2
            base = (chunk0 + ci) * _C
            pltpu.async_copy(snd_hbm.at[pl.ds(base, _C)], snd_v.at[i4], sem_in.at[i4])
            pltpu.async_copy(rcv_hbm.at[pl.ds(base, _C)], rcv_v.at[i4], sem_in.at[i4])
            pltpu.async_copy(
                w4_hbm.at[b, pl.ds((chunk0 + ci) * _C, _C), :],
                w_v.at[i2], sem_in.at[i4])

        def _wait_inputs(k):
            i4, i2 = k % 4, k % 2
            pltpu.make_async_copy(snd_hbm.at[pl.ds(0, _C)], snd_v.at[i4], sem_in.at[i4]).wait()
            pltpu.make_async_copy(rcv_hbm.at[pl.ds(0, _C)], rcv_v.at[i4], sem_in.at[i4]).wait()
            pltpu.make_async_copy(w4_hbm.at[0, pl.ds(0, _C), :],
                                  w_v.at[i2], sem_in.at[i4]).wait()

        def _issue_gather(k):
            i4, i2 = k % 4, k % 2
            pltpu.async_copy(nf_hbm.at[snd_v.at[i4]], fm_v.at[i2], sem_g.at[i2])

        def _wait_gather(k):
            i4, i2 = k % 4, k % 2
            pltpu.make_async_copy(nf_hbm.at[snd_v.at[i4]], fm_v.at[i2],
                                  sem_g.at[i2]).wait()

        def _issue_scatter(k):
            i4, i2 = k % 4, k % 2
            pltpu.async_copy(fm_v.at[i2], acc.at[rcv_v.at[i4]],
                             sem_sc.at[i2], add=True)

        def _wait_scatter(i2):
            pltpu.make_async_copy(fm_v.at[i2], acc.at[rcv_v.at[0]],
                                  sem_sc.at[i2]).wait()

        def _zb(i, carry):
            for j in range(8):
                fm_v[0, i, pl.ds(j * 16, 16)] = jnp.zeros((16,), jnp.float32)
            return carry
        lax.fori_loop(0, _C, _zb, 0)
        for z in range(_RPT // _C):
            pltpu.sync_copy(fm_v.at[0], acc.at[pl.ds(row0 + z * _C, _C), :])
        plsc.subcore_barrier()

        _issue_inputs(0, 0)
        _issue_inputs(1, 1)
        _wait_inputs(0)
        _issue_gather(0)

        def _compute(i2):
            def _edge(i, c2):
                for j in range(8):
                    sl = pl.ds(j * 16, 16)
                    fm_v[i2, i, sl] = fm_v[i2, i, sl] * w_v[i2, i, sl]
                return c2
            lax.fori_loop(0, _C, _edge, 0)

        def _outer(ci4, carry):
            for k in range(4):
                ci = ci4 * 4 + k
                i2 = k % 2

                _wait_inputs(k + 1)

                @pl.when(ci >= 1)
                def _():
                    _wait_scatter(1 - i2)

                _issue_gather(k + 1)
                _wait_gather(k)
                _compute(i2)
                _issue_scatter(k)
                _issue_inputs(ci + 2, k + 2)
            return carry
        lax.fori_loop(0, (n - 2) // 4, _outer, 0)

        _wait_inputs(1)
        _wait_scatter(1)
        _issue_gather(1)
        _wait_gather(0)
        _compute(0)
        _issue_scatter(0)
        _wait_scatter(0)
        _wait_gather(1)
        _compute(1)
        _issue_scatter(1)

        _wait_scatter((n - 1) % 2)
        plsc.subcore_barrier()

        for z in range(_RPT // _C):
            rr = row0 + z * _C
            pltpu.sync_copy(acc.at[pl.ds(rr, _C), :],
                            out_hbm.at[b, pl.ds(rr, _C), :])


def _sc_fn(node_feats, snd1, rcv1, w4):
    mesh = plsc.VectorSubcoreMesh(core_axis_name="c", subcore_axis_name="s")
    kern = functools.partial(
        pl.kernel,
        out_type=jax.ShapeDtypeStruct((4, _NP, 128), jnp.float32),
        mesh=mesh,
        scratch_types=[
            pltpu.VMEM_SHARED((_NP, 128), jnp.float32),
            pltpu.VMEM((4, _C), jnp.int32),
            pltpu.VMEM((4, _C), jnp.int32),
            pltpu.VMEM((2, _C, 128), jnp.float32),
            pltpu.VMEM((2, _C, 128), jnp.float32),
            pltpu.SemaphoreType.DMA((4,)),
            pltpu.SemaphoreType.DMA((2,)),
            pltpu.SemaphoreType.DMA((2,)),
        ],
    )(_sc_body)
    return kern(node_feats, snd1, rcv1, w4)


def kernel(vectors, node_feats, radial_embedding, senders, receivers,
           W0, W1, W2, W3):
    w4 = _coef_fn(vectors, radial_embedding, W0, W1, W2, W3)
    out4 = _sc_fn(node_feats, senders, receivers, w4)[:, :N_NODES]
    out_s = out4[0]
    out_v = out4[1:].transpose(1, 2, 0).reshape(N_NODES, 3 * D_FEAT)
    return jnp.concatenate([out_s, out_v], axis=1)


# speedup vs baseline: 3.6037x; 2.1069x over previous
import functools
import math

import jax
import jax.numpy as jnp
from jax import lax
from jax.experimental import pallas as pl
from jax.experimental.pallas import tpu as pltpu
from jax.experimental.pallas import tpu_sc as plsc

N_NODES = 10000
N_EDGES = 320000
D_FEAT = 128
AVG_NUM_NEIGHBORS = 32.0


_B = 8000


def _coef_body(vec_ref, rad_ref, w0, w1, w2, w3, w4_ref):
    prec = jax.lax.Precision.DEFAULT
    x = rad_ref[...]
    h = jax.nn.silu(jnp.dot(x, w0[...], precision=prec,
                            preferred_element_type=jnp.float32) * (8.0 ** -0.5))
    h = jax.nn.silu(jnp.dot(h, w1[...], precision=prec,
                            preferred_element_type=jnp.float32) * 0.125)
    h = jax.nn.silu(jnp.dot(h, w2[...], precision=prec,
                            preferred_element_type=jnp.float32) * 0.125)
    mix = jnp.dot(h, w3[...], precision=prec,
                  preferred_element_type=jnp.float32)
    mix = mix * (0.125 / math.sqrt(AVG_NUM_NEIGHBORS))

    v = vec_ref[...]
    inv = jax.lax.rsqrt(jnp.sum(v * v, axis=1, keepdims=True) + 1e-12)
    sh = v * inv * math.sqrt(3.0)

    cv = mix[:, D_FEAT:]
    w4_ref[0] = mix[:, :D_FEAT]
    for k in range(3):
        w4_ref[k + 1] = cv * sh[:, k:k + 1]


def _coef_fn(vectors, radial, W0, W1, W2, W3):
    grid = (N_EDGES // _B,)
    return pl.pallas_call(
        _coef_body,
        grid=grid,
        in_specs=[
            pl.BlockSpec((_B, 3), lambda i: (i, 0)),
            pl.BlockSpec((_B, 8), lambda i: (i, 0)),
            pl.BlockSpec((8, 64), lambda i: (0, 0)),
            pl.BlockSpec((64, 64), lambda i: (0, 0)),
            pl.BlockSpec((64, 64), lambda i: (0, 0)),
            pl.BlockSpec((64, 256), lambda i: (0, 0)),
        ],
        out_specs=pl.BlockSpec((4, _B, 128), lambda i: (0, i, 0)),
        out_shape=jax.ShapeDtypeStruct((4, N_EDGES, 128), jnp.float32),
    )(vectors, radial, W0, W1, W2, W3)


_NS = 16
_EPT = N_EDGES // _NS
_C = 80
_NCHUNK = _EPT // _C
_NP = 10240
_RPT = _NP // _NS


def _sc_body(nf_hbm, snd_hbm, rcv_hbm, w4_hbm, out_hbm,
             acc, snd_v, rcv_v, fm_v, w_v, sem_in, sem_g, sem_sc):
    cid = lax.axis_index("c")
    sid = lax.axis_index("s")
    chunk0 = sid * _NCHUNK
    row0 = sid * _RPT
    n = _NCHUNK

    for r in range(2):
        b = cid * 2 + r

        def _issue_inputs(ci, k):
            i4, i2 = k % 4, k % 2
            base = (chunk0 + ci) * _C
            pltpu.async_copy(snd_hbm.at[pl.ds(base, _C)], snd_v.at[i4], sem_in.at[i4])
            pltpu.async_copy(rcv_hbm.at[pl.ds(base, _C)], rcv_v.at[i4], sem_in.at[i4])
            pltpu.async_copy(
                w4_hbm.at[b, pl.ds((chunk0 + ci) * _C, _C), :],
                w_v.at[i2], sem_in.at[i4])

        def _wait_inputs(k):
            i4, i2 = k % 4, k % 2
            pltpu.make_async_copy(snd_hbm.at[pl.ds(0, _C)], snd_v.at[i4], sem_in.at[i4]).wait()
            pltpu.make_async_copy(rcv_hbm.at[pl.ds(0, _C)], rcv_v.at[i4], sem_in.at[i4]).wait()
            pltpu.make_async_copy(w4_hbm.at[0, pl.ds(0, _C), :],
                                  w_v.at[i2], sem_in.at[i4]).wait()

        def _issue_gather(k):
            i4, i2 = k % 4, k % 2
            pltpu.async_copy(nf_hbm.at[snd_v.at[i4]], fm_v.at[i2], sem_g.at[i2])

        def _wait_gather(k):
            i4, i2 = k % 4, k % 2
            pltpu.make_async_copy(nf_hbm.at[snd_v.at[i4]], fm_v.at[i2],
                                  sem_g.at[i2]).wait()

        def _issue_scatter(k):
            i4, i2 = k % 4, k % 2
            pltpu.async_copy(fm_v.at[i2], acc.at[rcv_v.at[i4]],
                             sem_sc.at[i2], add=True)

        def _wait_scatter(i2):
            pltpu.make_async_copy(fm_v.at[i2], acc.at[rcv_v.at[0]],
                                  sem_sc.at[i2]).wait()

        def _zb(i, carry):
            for j in range(8):
                fm_v[0, i, pl.ds(j * 16, 16)] = jnp.zeros((16,), jnp.float32)
            return carry
        lax.fori_loop(0, _C, _zb, 0)
        for z in range(_RPT // _C):
            pltpu.sync_copy(fm_v.at[0], acc.at[pl.ds(row0 + z * _C, _C), :])
        plsc.subcore_barrier()

        _issue_inputs(0, 0)
        _issue_inputs(1, 1)
        _wait_inputs(0)
        _issue_gather(0)

        def _compute(i2):
            def _edge(i, c2):
                for j in range(8):
                    sl = pl.ds(j * 16, 16)
                    fm_v[i2, i, sl] = fm_v[i2, i, sl] * w_v[i2, i, sl]
                return c2
            lax.fori_loop(0, _C, _edge, 0)

        def _outer(ci4, carry):
            for k in range(4):
                ci = ci4 * 4 + k
                i2 = k % 2

                _wait_inputs(k + 1)

                @pl.when(ci >= 1)
                def _():
                    _wait_scatter(1 - i2)

                _issue_gather(k + 1)
                _wait_gather(k)
                _compute(i2)
                _issue_scatter(k)
                _issue_inputs(ci + 2, k + 2)
            return carry
        lax.fori_loop(0, (n - 2) // 4, _outer, 0)

        _wait_inputs(1)
        _wait_scatter(1)
        _issue_gather(1)
        _wait_gather(0)
        _compute(0)
        _issue_scatter(0)
        _wait_scatter(0)
        _wait_gather(1)
        _compute(1)
        _issue_scatter(1)

        _wait_scatter((n - 1) % 2)
        plsc.subcore_barrier()

        for z in range(_RPT // _C):
            rr = row0 + z * _C
            pltpu.sync_copy(acc.at[pl.ds(rr, _C), :],
                            out_hbm.at[b, pl.ds(rr, _C), :])


def _sc_fn(node_feats, snd1, rcv1, w4):
    mesh = plsc.VectorSubcoreMesh(core_axis_name="c", subcore_axis_name="s")
    kern = functools.partial(
        pl.kernel,
        out_type=jax.ShapeDtypeStruct((4, _NP, 128), jnp.float32),
        mesh=mesh,
        scratch_types=[
            pltpu.VMEM_SHARED((_NP, 128), jnp.float32),
            pltpu.VMEM((4, _C), jnp.int32),
            pltpu.VMEM((4, _C), jnp.int32),
            pltpu.VMEM((2, _C, 128), jnp.float32),
            pltpu.VMEM((2, _C, 128), jnp.float32),
            pltpu.SemaphoreType.DMA((4,)),
            pltpu.SemaphoreType.DMA((2,)),
            pltpu.SemaphoreType.DMA((2,)),
        ],
    )(_sc_body)
    return kern(node_feats, snd1, rcv1, w4)


def kernel(vectors, node_feats, radial_embedding, senders, receivers,
           W0, W1, W2, W3):
    w4 = _coef_fn(vectors, radial_embedding, W0, W1, W2, W3)
    out4 = _sc_fn(node_feats, senders, receivers, w4)[:, :N_NODES]
    out_s = out4[0]
    out_v = out4[1:].transpose(1, 2, 0).reshape(N_NODES, 3 * D_FEAT)
    return jnp.concatenate([out_s, out_v], axis=1)
